# Initial kernel scaffold; baseline (speedup 1.0000x reference)
import jax
import jax.numpy as jnp
from jax.experimental import pallas as pl


def kernel(segment_ids, h, W0, b0):
    raise NotImplementedError("write your pallas kernel here")


# speedup vs baseline: 3.5934x; 3.5934x over previous
import functools

import jax
import jax.numpy as jnp
from jax import lax
from jax.experimental import pallas as pl
from jax.experimental.pallas import tpu as pltpu
from jax.experimental.pallas import tpu_sc as plsc

_N_NODES = 100000
_D = 128
_G = 64

_NC = 2
_NS = 16
_NW = _NC * _NS

_CHUNK = 80
_NCHUNKS = _N_NODES // _CHUNK
_CPW = _NCHUNKS // _NW
_EXTRA = _NCHUNKS - _CPW * _NW
_MAXC = _CPW + 1


def _segment_sum_sc(seg2d, h, zacc):
  mesh = plsc.VectorSubcoreMesh(core_axis_name="c", subcore_axis_name="s")

  @functools.partial(
      pl.kernel,
      out_type=jax.ShapeDtypeStruct((_NC, _G, _D), jnp.float32),
      mesh=mesh,
      scratch_types=[
          pltpu.VMEM((_CHUNK,), jnp.int32),
          pltpu.VMEM((_CHUNK, _D), jnp.float32),
          pltpu.VMEM((_G, _D), jnp.float32),
          pltpu.VMEM_SHARED((_G, _D), jnp.float32),
      ],
  )
  def k(seg_hbm, h_hbm, z_hbm, out_hbm, idx_v, buf_v, obuf_v, acc_sh):
    cid = lax.axis_index("c")
    sid = lax.axis_index("s")
    wid = sid * _NC + cid
    nmine = jnp.where(wid < _EXTRA, _CPW + 1, _CPW)
    start = wid * _CPW + jnp.minimum(wid, _EXTRA)

    @pl.when(sid == 0)
    def _():
      pltpu.sync_copy(z_hbm, acc_sh)

    plsc.subcore_barrier()

    def body(j, carry):
      @pl.when(j < nmine)
      def _():
        row0 = (start + j) * _CHUNK
        pltpu.sync_copy(seg_hbm.at[pl.ds(row0, _CHUNK)], idx_v)
        pltpu.sync_copy(h_hbm.at[pl.ds(row0, _CHUNK)], buf_v)
        pltpu.sync_copy(buf_v, acc_sh.at[idx_v], add=True)
      return carry

    lax.fori_loop(0, _MAXC, body, 0)

    plsc.subcore_barrier()

    @pl.when(sid == 0)
    def _():
      pltpu.sync_copy(acc_sh, obuf_v)
      pltpu.sync_copy(obuf_v, out_hbm.at[cid])

  return k(seg2d, h, zacc)


def _readout_tc(partials, W0, b0):
  def mm(p_ref, w_ref, b_ref, o_ref):
    pooled = p_ref[0] + p_ref[1]
    o_ref[...] = lax.dot_general(
        pooled, w_ref[...], (((1,), (1,)), ((), ())),
        preferred_element_type=jnp.float32) + b_ref[...]

  return pl.pallas_call(
      mm,
      out_shape=jax.ShapeDtypeStruct((_G, _D), jnp.float32),
  )(partials, W0, b0.reshape(1, _D))


def kernel(segment_ids, h, W0, b0):
  zacc = jnp.zeros((_G, _D), jnp.float32)
  partials = _segment_sum_sc(segment_ids, h, zacc)
  return _readout_tc(partials, W0, b0)


# speedup vs baseline: 4.7158x; 1.3123x over previous
import functools

import jax
import jax.numpy as jnp
from jax import lax
from jax.experimental import pallas as pl
from jax.experimental.pallas import tpu as pltpu
from jax.experimental.pallas import tpu_sc as plsc

_N_NODES = 100000
_D = 128
_G = 64

_NC = 2
_NS = 16
_NW = _NC * _NS

_CHUNK = 80
_NCHUNKS = _N_NODES // _CHUNK
_CPW = _NCHUNKS // _NW
_EXTRA = _NCHUNKS - _CPW * _NW
_MAXC = _CPW + 1


def _segment_sum_sc(seg2d, h, zacc):
  mesh = plsc.VectorSubcoreMesh(core_axis_name="c", subcore_axis_name="s")

  @functools.partial(
      pl.kernel,
      out_type=jax.ShapeDtypeStruct((_NC, _G, _D), jnp.float32),
      mesh=mesh,
      scratch_types=[
          pltpu.VMEM((_CHUNK,), jnp.int32),
          pltpu.VMEM((_CHUNK,), jnp.int32),
          pltpu.VMEM((_CHUNK,), jnp.int32),
          pltpu.VMEM((_CHUNK, _D), jnp.float32),
          pltpu.VMEM((_CHUNK, _D), jnp.float32),
          pltpu.VMEM((_CHUNK, _D), jnp.float32),
          pltpu.VMEM((_G, _D), jnp.float32),
          pltpu.VMEM_SHARED((_G, _D), jnp.float32),
          pltpu.SemaphoreType.DMA,
          pltpu.SemaphoreType.DMA,
          pltpu.SemaphoreType.DMA,
          pltpu.SemaphoreType.DMA,
          pltpu.SemaphoreType.DMA,
          pltpu.SemaphoreType.DMA,
          pltpu.SemaphoreType.DMA,
          pltpu.SemaphoreType.DMA,
          pltpu.SemaphoreType.DMA,
      ],
  )
  def k(seg_hbm, h_hbm, z_hbm, out_hbm,
        i0, i1, i2, hb0, hb1, hb2, obuf_v, acc_sh,
        is0, is1, is2, hs0, hs1, hs2, ss0, ss1, ss2):
    idx = (i0, i1, i2)
    hbuf = (hb0, hb1, hb2)
    isem = (is0, is1, is2)
    hsem = (hs0, hs1, hs2)
    ssem = (ss0, ss1, ss2)

    cid = lax.axis_index("c")
    sid = lax.axis_index("s")
    wid = sid * _NC + cid
    nmine = jnp.where(wid < _EXTRA, _CPW + 1, _CPW)
    start = wid * _CPW + jnp.minimum(wid, _EXTRA)

    def g_desc(j, b):
      row0 = (start + j) * _CHUNK
      return (pltpu.make_async_copy(seg_hbm.at[pl.ds(row0, _CHUNK)],
                                    idx[b], isem[b]),
              pltpu.make_async_copy(h_hbm.at[pl.ds(row0, _CHUNK)],
                                    hbuf[b], hsem[b]))

    def s_desc(b):
      return pltpu.make_async_copy(hbuf[b], acc_sh.at[idx[b]], ssem[b])

    @pl.when(sid == 0)
    def _():
      pltpu.sync_copy(z_hbm, acc_sh)

    plsc.subcore_barrier()

    for j in (0, 1):
      di, dh = g_desc(j, j % 3)
      di.start()
      dh.start()

    for j in range(_MAXC):
      b = j % 3
      di, dh = g_desc(j, b)

      @pl.when(j < nmine)
      def _(di=di, dh=dh, b=b):
        di.wait()
        dh.wait()
        s_desc(b).start(add=True)

      if j == 0:
        di2, dh2 = g_desc(2, 2)
        di2.start()
        dh2.start()
      else:
        bp = (j - 1) % 3
        di2, dh2 = g_desc(j + 2, (j + 2) % 3)

        @pl.when(j + 2 < nmine)
        def _(bp=bp, di2=di2, dh2=dh2):
          s_desc(bp).wait()
          di2.start()
          dh2.start()

    for b in range(3):
      s_desc(b).wait()

    plsc.subcore_barrier()

    @pl.when(sid == 0)
    def _():
      pltpu.sync_copy(acc_sh, obuf_v)
      pltpu.sync_copy(obuf_v, out_hbm.at[cid])

  return k(seg2d, h, zacc)


def _readout_tc(partials, W0, b0):
  def mm(p_ref, w_ref, b_ref, o_ref):
    pooled = p_ref[0] + p_ref[1]
    o_ref[...] = lax.dot_general(
        pooled, w_ref[...], (((1,), (1,)), ((), ())),
        preferred_element_type=jnp.float32) + b_ref[...]

  return pl.pallas_call(
      mm,
      out_shape=jax.ShapeDtypeStruct((_G, _D), jnp.float32),
  )(partials, W0, b0.reshape(1, _D))


def kernel(segment_ids, h, W0, b0):
  zacc = jnp.zeros((_G, _D), jnp.float32)
  partials = _segment_sum_sc(segment_ids, h, zacc)
  return _readout_tc(partials, W0, b0)
